# baseline (speedup 1.0000x reference)
import functools

import jax
import jax.numpy as jnp
from jax import lax
from jax.experimental import pallas as pl
from jax.experimental.pallas import tpu as pltpu
from jax.experimental.pallas import tpu_sc as plsc

_N_NODES = 10000
_D = 128
_N_EDGES = 320000
_BN_EPS = 1e-5

_NC = 2
_NS = 16
_NW = _NC * _NS
_EPW = _N_EDGES // _NW
_K = 80
_NCHUNK = _EPW // _K
_NPAD = 10240
_RPT = _NPAD // _NS


@functools.partial(
    pl.kernel,
    out_type=jax.ShapeDtypeStruct((_NC, _NPAD, _D), jnp.float32),
    mesh=plsc.VectorSubcoreMesh(core_axis_name="c", subcore_axis_name="s"),
    scratch_types=[
        pltpu.VMEM((_K, _D), jnp.float32),
        pltpu.VMEM((_K, _D), jnp.float32),
        pltpu.VMEM((2, _K), jnp.int32),
        pltpu.VMEM((2, _K), jnp.int32),
        pltpu.VMEM((2, _K), jnp.int32),
        pltpu.VMEM((2, _K), jnp.int32),
        pltpu.VMEM_SHARED((_NPAD, _D), jnp.float32),
        pltpu.SemaphoreType.DMA,
        pltpu.SemaphoreType.DMA,
        pltpu.SemaphoreType.DMA,
        pltpu.SemaphoreType.DMA,
        pltpu.SemaphoreType.DMA,
        pltpu.SemaphoreType.DMA,
    ],
)
def _sc_segment_sum(x_hbm, e_hbm, out_hbm,
                    rows0, rows1, idx0, idx1, idx2, idx3, agg_sh,
                    sem0, sem1, sems0, sems1, semi0, semi1):
    cid = lax.axis_index("c")
    sid = lax.axis_index("s")
    wid = cid * _NS + sid

    rows = (rows0, rows1)
    idx = (idx0, idx1, idx2, idx3)
    gsem = (sem0, sem1)
    ssem = (sems0, sems1)
    isem = (semi0, semi1)

    zeros16 = jnp.zeros((16,), jnp.float32)

    @pl.loop(0, _K)
    def _(i):
        @pl.loop(0, _D // 16)
        def _(j):
            rows0.at[i, pl.ds(j * 16, 16)][...] = zeros16

    @pl.loop(0, _RPT // _K)
    def _(r):
        pltpu.sync_copy(rows0, agg_sh.at[pl.ds(sid * _RPT + r * _K, _K)])

    plsc.subcore_barrier()

    pltpu.sync_copy(e_hbm.at[wid, 0], idx[0])
    pltpu.async_copy(x_hbm.at[idx[0].at[0]], rows[0], gsem[0])
    pltpu.async_copy(e_hbm.at[wid, 1], idx[1], isem[1])

    def _step(i, k, first=False, has_next=True, has_next2=True):
        p = k % 2
        q = 1 - p
        s = k % 4
        pltpu.make_async_copy(x_hbm.at[idx[s].at[0]], rows[p], gsem[p]).wait()
        pltpu.async_copy(rows[p], agg_sh.at[idx[s].at[1]], ssem[p], add=True)
        if has_next:
            if not first:
                pltpu.make_async_copy(
                    rows[q], agg_sh.at[idx[(s + 3) % 4].at[1]], ssem[q]).wait()
            sn = (s + 1) % 4
            pltpu.make_async_copy(e_hbm.at[wid, i + 1], idx[sn],
                                  isem[sn % 2]).wait()
            pltpu.async_copy(x_hbm.at[idx[sn].at[0]], rows[q], gsem[q])
        if has_next2:
            sn2 = (s + 2) % 4
            pltpu.async_copy(e_hbm.at[wid, i + 2], idx[sn2], isem[sn2 % 2])

    _step(0, 0, first=True)
    _step(1, 1)
    _step(2, 2)
    _step(3, 3)

    @pl.loop(1, (_NCHUNK - 5) // 4)
    def _(g):
        _step(4 * g, 0)
        _step(4 * g + 1, 1)
        _step(4 * g + 2, 2)
        _step(4 * g + 3, 3)

    _step(_NCHUNK - 5, 0)
    _step(_NCHUNK - 4, 1)
    _step(_NCHUNK - 3, 2)
    _step(_NCHUNK - 2, 3, has_next2=False)
    _step(_NCHUNK - 1, 0, has_next=False, has_next2=False)

    pltpu.make_async_copy(rows[1], agg_sh.at[idx[3].at[1]], ssem[1]).wait()
    pltpu.make_async_copy(rows[0], agg_sh.at[idx[0].at[1]], ssem[0]).wait()

    plsc.subcore_barrier()
    pltpu.sync_copy(agg_sh.at[pl.ds(sid * _RPT, _RPT)],
                    out_hbm.at[cid].at[pl.ds(sid * _RPT, _RPT)])


def _mlp_body(x_ref, agg_ref, w1_ref, g1_ref, b1_ref, w2_ref, g2_ref, b2_ref,
              eps_ref, o_ref):
    h = ((1.0 + eps_ref[0, 0]) * x_ref[...]
         + agg_ref[0, :_N_NODES, :] + agg_ref[1, :_N_NODES, :])
    dn = (((1,), (1,)), ((), ()))
    h = lax.dot_general(h, w1_ref[...], dn,
                        preferred_element_type=jnp.float32,
                        precision=lax.Precision.HIGHEST)
    mu = jnp.mean(h, axis=0, keepdims=True)
    var = jnp.mean((h - mu) ** 2, axis=0, keepdims=True)
    h = g1_ref[...] * (h - mu) * lax.rsqrt(var + _BN_EPS) + b1_ref[...]
    h = jnp.maximum(h, 0.0)
    h = lax.dot_general(h, w2_ref[...], dn,
                        preferred_element_type=jnp.float32,
                        precision=lax.Precision.HIGHEST)
    mu = jnp.mean(h, axis=0, keepdims=True)
    var = jnp.mean((h - mu) ** 2, axis=0, keepdims=True)
    h = g2_ref[...] * (h - mu) * lax.rsqrt(var + _BN_EPS) + b2_ref[...]
    o_ref[...] = jnp.maximum(h, 0.0)


@jax.jit
def kernel(x, edge_index, batch, W1, gamma1, beta1, W2, gamma2, beta2, eps):
    del batch
    ei = edge_index.astype(jnp.int32)
    e = jnp.stack([ei[0].reshape(_NW, _NCHUNK, _K),
                   ei[1].reshape(_NW, _NCHUNK, _K)], axis=2)
    agg = _sc_segment_sum(x, e)
    return pl.pallas_call(
        _mlp_body,
        out_shape=jax.ShapeDtypeStruct((_N_NODES, _D), jnp.float32),
    )(x, agg, W1, gamma1.reshape(1, _D), beta1.reshape(1, _D),
      W2, gamma2.reshape(1, _D), beta2.reshape(1, _D),
      eps.reshape(1, 1).astype(jnp.float32))


# speedup vs baseline: 1.1239x; 1.1239x over previous
import functools

import jax
import jax.numpy as jnp
from jax import lax
from jax.experimental import pallas as pl
from jax.experimental.pallas import tpu as pltpu
from jax.experimental.pallas import tpu_sc as plsc

_N_NODES = 10000
_D = 128
_N_EDGES = 320000
_BN_EPS = 1e-5

_NC = 2
_NS = 16
_NW = _NC * _NS
_EPW = _N_EDGES // _NW
_K = 80
_NCHUNK = _EPW // _K
_NPAD = 10240
_RPT = _NPAD // _NS


@functools.partial(
    pl.kernel,
    out_type=jax.ShapeDtypeStruct((_NC, _NPAD, _D), jnp.float32),
    mesh=plsc.VectorSubcoreMesh(core_axis_name="c", subcore_axis_name="s"),
    scratch_types=[
        pltpu.VMEM((_K, _D), jnp.float32),
        pltpu.VMEM((_K, _D), jnp.float32),
        pltpu.VMEM((4, _K), jnp.int32),
        pltpu.VMEM((4, _K), jnp.int32),
        pltpu.VMEM_SHARED((_NPAD, _D), jnp.float32),
        pltpu.SemaphoreType.DMA,
        pltpu.SemaphoreType.DMA,
        pltpu.SemaphoreType.DMA,
        pltpu.SemaphoreType.DMA,
        pltpu.SemaphoreType.DMA,
        pltpu.SemaphoreType.DMA,
    ],
)
def _sc_segment_sum(x_hbm, src_hbm, dst_hbm, out_hbm,
                    rows0, rows1, sidx, didx, agg_sh,
                    sem0, sem1, sems0, sems1, semi0, semi1):
    cid = lax.axis_index("c")
    sid = lax.axis_index("s")
    wid = cid * _NS + sid

    rows = (rows0, rows1)
    gsem = (sem0, sem1)
    ssem = (sems0, sems1)
    isem = (semi0, semi1)

    def _load_idx(c, s, sem=None):
        sslc = src_hbm.at[wid].at[pl.ds(c, 1)]
        dslc = dst_hbm.at[wid].at[pl.ds(c, 1)]
        if sem is None:
            pltpu.sync_copy(sslc, sidx.at[pl.ds(s, 1)])
            pltpu.sync_copy(dslc, didx.at[pl.ds(s, 1)])
        else:
            pltpu.async_copy(sslc, sidx.at[pl.ds(s, 1)], sem)
            pltpu.async_copy(dslc, didx.at[pl.ds(s, 1)], sem)

    def _wait_idx(c, s, sem):
        pltpu.make_async_copy(src_hbm.at[wid].at[pl.ds(c, 1)],
                              sidx.at[pl.ds(s, 1)], sem).wait()
        pltpu.make_async_copy(dst_hbm.at[wid].at[pl.ds(c, 1)],
                              didx.at[pl.ds(s, 1)], sem).wait()

    zeros16 = jnp.zeros((16,), jnp.float32)

    @pl.loop(0, _K)
    def _(i):
        @pl.loop(0, _D // 16)
        def _(j):
            rows0.at[i, pl.ds(j * 16, 16)][...] = zeros16

    @pl.loop(0, _RPT // _K)
    def _(r):
        pltpu.sync_copy(rows0, agg_sh.at[pl.ds(sid * _RPT + r * _K, _K)])

    plsc.subcore_barrier()

    _load_idx(0, 0)
    pltpu.async_copy(x_hbm.at[sidx.at[0]], rows[0], gsem[0])
    _load_idx(1, 1, isem[1])

    def _step(i, k, first=False, has_next=True, has_next2=True):
        p = k % 2
        q = 1 - p
        s = k % 4
        pltpu.make_async_copy(x_hbm.at[sidx.at[s]], rows[p], gsem[p]).wait()
        pltpu.async_copy(rows[p], agg_sh.at[didx.at[s]], ssem[p], add=True)
        if has_next:
            if not first:
                pltpu.make_async_copy(
                    rows[q], agg_sh.at[didx.at[(s + 3) % 4]], ssem[q]).wait()
            sn = (s + 1) % 4
            _wait_idx(i + 1, sn, isem[sn % 2])
            pltpu.async_copy(x_hbm.at[sidx.at[sn]], rows[q], gsem[q])
        if has_next2:
            sn2 = (s + 2) % 4
            _load_idx(i + 2, sn2, isem[sn2 % 2])

    _step(0, 0, first=True)
    _step(1, 1)
    _step(2, 2)
    _step(3, 3)

    @pl.loop(1, (_NCHUNK - 5) // 4)
    def _(g):
        _step(4 * g, 0)
        _step(4 * g + 1, 1)
        _step(4 * g + 2, 2)
        _step(4 * g + 3, 3)

    _step(_NCHUNK - 5, 0)
    _step(_NCHUNK - 4, 1)
    _step(_NCHUNK - 3, 2)
    _step(_NCHUNK - 2, 3, has_next2=False)
    _step(_NCHUNK - 1, 0, has_next=False, has_next2=False)

    pltpu.make_async_copy(rows[1], agg_sh.at[didx.at[3]], ssem[1]).wait()
    pltpu.make_async_copy(rows[0], agg_sh.at[didx.at[0]], ssem[0]).wait()

    plsc.subcore_barrier()
    pltpu.sync_copy(agg_sh.at[pl.ds(sid * _RPT, _RPT)],
                    out_hbm.at[cid].at[pl.ds(sid * _RPT, _RPT)])


def _mlp_body(x_ref, agg_ref, w1_ref, g1_ref, b1_ref, w2_ref, g2_ref, b2_ref,
              eps_ref, o_ref):
    h = ((1.0 + eps_ref[0, 0]) * x_ref[...]
         + agg_ref[0, :_N_NODES, :] + agg_ref[1, :_N_NODES, :])
    dn = (((1,), (1,)), ((), ()))
    h = lax.dot_general(h, w1_ref[...], dn,
                        preferred_element_type=jnp.float32)
    mu = jnp.mean(h, axis=0, keepdims=True)
    var = jnp.mean((h - mu) ** 2, axis=0, keepdims=True)
    h = g1_ref[...] * (h - mu) * lax.rsqrt(var + _BN_EPS) + b1_ref[...]
    h = jnp.maximum(h, 0.0)
    h = lax.dot_general(h, w2_ref[...], dn,
                        preferred_element_type=jnp.float32)
    mu = jnp.mean(h, axis=0, keepdims=True)
    var = jnp.mean((h - mu) ** 2, axis=0, keepdims=True)
    h = g2_ref[...] * (h - mu) * lax.rsqrt(var + _BN_EPS) + b2_ref[...]
    o_ref[...] = jnp.maximum(h, 0.0)


@jax.jit
def kernel(x, edge_index, batch, W1, gamma1, beta1, W2, gamma2, beta2, eps):
    del batch
    ei = edge_index.astype(jnp.int32)
    src = ei[0].reshape(_NW, _NCHUNK, _K)
    dst = ei[1].reshape(_NW, _NCHUNK, _K)
    agg = _sc_segment_sum(x, src, dst)
    return pl.pallas_call(
        _mlp_body,
        out_shape=jax.ShapeDtypeStruct((_N_NODES, _D), jnp.float32),
    )(x, agg, W1, gamma1.reshape(1, _D), beta1.reshape(1, _D),
      W2, gamma2.reshape(1, _D), beta2.reshape(1, _D),
      eps.reshape(1, 1).astype(jnp.float32))


# speedup vs baseline: 1.1299x; 1.0053x over previous
import functools

import jax
import jax.numpy as jnp
from jax import lax
from jax.experimental import pallas as pl
from jax.experimental.pallas import tpu as pltpu
from jax.experimental.pallas import tpu_sc as plsc

_N_NODES = 10000
_D = 128
_N_EDGES = 320000
_BN_EPS = 1e-5

_NC = 2
_NS = 16
_NW = _NC * _NS
_EPW = _N_EDGES // _NW
_K = 80
_NCHUNK = _EPW // _K
_NPAD = 10240
_RPT = _NPAD // _NS


@functools.partial(
    pl.kernel,
    out_type=jax.ShapeDtypeStruct((_NC, _NPAD, _D), jnp.float32),
    mesh=plsc.VectorSubcoreMesh(core_axis_name="c", subcore_axis_name="s"),
    scratch_types=[
        pltpu.VMEM((_K, _D), jnp.float32),
        pltpu.VMEM((_K, _D), jnp.float32),
        pltpu.VMEM((4, _K), jnp.int32),
        pltpu.VMEM((4, _K), jnp.int32),
        pltpu.VMEM_SHARED((_NPAD, _D), jnp.float32),
        pltpu.SemaphoreType.DMA,
        pltpu.SemaphoreType.DMA,
        pltpu.SemaphoreType.DMA,
        pltpu.SemaphoreType.DMA,
        pltpu.SemaphoreType.DMA,
        pltpu.SemaphoreType.DMA,
    ],
)
def _sc_segment_sum(x_hbm, src_hbm, dst_hbm, out_hbm,
                    rows0, rows1, sidx, didx, agg_sh,
                    sem0, sem1, sems0, sems1, semi0, semi1):
    cid = lax.axis_index("c")
    sid = lax.axis_index("s")
    wid = cid * _NS + sid

    rows = (rows0, rows1)
    gsem = (sem0, sem1)
    ssem = (sems0, sems1)
    isem = (semi0, semi1)

    def _load_idx(c, s, sem=None):
        sslc = src_hbm.at[wid].at[pl.ds(c, 1)]
        dslc = dst_hbm.at[wid].at[pl.ds(c, 1)]
        if sem is None:
            pltpu.sync_copy(sslc, sidx.at[pl.ds(s, 1)])
            pltpu.sync_copy(dslc, didx.at[pl.ds(s, 1)])
        else:
            pltpu.async_copy(sslc, sidx.at[pl.ds(s, 1)], sem)
            pltpu.async_copy(dslc, didx.at[pl.ds(s, 1)], sem)

    def _wait_idx(c, s, sem):
        pltpu.make_async_copy(src_hbm.at[wid].at[pl.ds(c, 1)],
                              sidx.at[pl.ds(s, 1)], sem).wait()
        pltpu.make_async_copy(dst_hbm.at[wid].at[pl.ds(c, 1)],
                              didx.at[pl.ds(s, 1)], sem).wait()

    zeros16 = jnp.zeros((16,), jnp.float32)

    @pl.loop(0, _K)
    def _(i):
        @pl.loop(0, _D // 16)
        def _(j):
            rows0.at[i, pl.ds(j * 16, 16)][...] = zeros16

    @pl.loop(0, _RPT // _K)
    def _(r):
        pltpu.sync_copy(rows0, agg_sh.at[pl.ds(sid * _RPT + r * _K, _K)])

    plsc.subcore_barrier()

    _load_idx(0, 0)
    pltpu.async_copy(x_hbm.at[sidx.at[0]], rows[0], gsem[0])
    _load_idx(1, 1, isem[1])

    def _step(i, k, first=False, has_next=True, has_next2=True):
        p = k % 2
        q = 1 - p
        s = k % 4
        pltpu.make_async_copy(x_hbm.at[sidx.at[s]], rows[p], gsem[p]).wait()
        pltpu.async_copy(rows[p], agg_sh.at[didx.at[s]], ssem[p], add=True)
        if has_next:
            if not first:
                pltpu.make_async_copy(
                    rows[q], agg_sh.at[didx.at[(s + 3) % 4]], ssem[q]).wait()
            sn = (s + 1) % 4
            _wait_idx(i + 1, sn, isem[sn % 2])
            pltpu.async_copy(x_hbm.at[sidx.at[sn]], rows[q], gsem[q])
        if has_next2:
            sn2 = (s + 2) % 4
            _load_idx(i + 2, sn2, isem[sn2 % 2])

    _step(0, 0, first=True)
    _step(1, 1)
    _step(2, 2)
    _step(3, 3)

    @pl.loop(1, (_NCHUNK - 5) // 4)
    def _(g):
        _step(4 * g, 0)
        _step(4 * g + 1, 1)
        _step(4 * g + 2, 2)
        _step(4 * g + 3, 3)

    _step(_NCHUNK - 5, 0)
    _step(_NCHUNK - 4, 1)
    _step(_NCHUNK - 3, 2)
    _step(_NCHUNK - 2, 3, has_next2=False)
    _step(_NCHUNK - 1, 0, has_next=False, has_next2=False)

    pltpu.make_async_copy(rows[1], agg_sh.at[didx.at[3]], ssem[1]).wait()
    pltpu.make_async_copy(rows[0], agg_sh.at[didx.at[0]], ssem[0]).wait()

    plsc.subcore_barrier()
    pltpu.sync_copy(agg_sh.at[pl.ds(sid * _RPT, _RPT)],
                    out_hbm.at[cid].at[pl.ds(sid * _RPT, _RPT)])


def _mlp_body(x_ref, agg_ref, w1_ref, g1_ref, b1_ref, w2_ref, g2_ref, b2_ref,
              eps_ref, o_ref):
    dn = (((1,), (1,)), ((), ()))
    ones = jnp.ones((1, _N_NODES), jnp.float32)

    def _bn_relu(h, g_ref, b_ref):
        dnc = (((1,), (0,)), ((), ()))
        s = lax.dot_general(ones, h, dnc, preferred_element_type=jnp.float32)
        q = lax.dot_general(ones, h * h, dnc,
                            preferred_element_type=jnp.float32)
        mu = s * (1.0 / _N_NODES)
        var = q * (1.0 / _N_NODES) - mu * mu
        scale = g_ref[...] * lax.rsqrt(var + _BN_EPS)
        return jnp.maximum(scale * (h - mu) + b_ref[...], 0.0)

    h = ((1.0 + eps_ref[0, 0]) * x_ref[...]
         + agg_ref[0, :_N_NODES, :] + agg_ref[1, :_N_NODES, :])
    h = lax.dot_general(h, w1_ref[...], dn,
                        preferred_element_type=jnp.float32)
    h = _bn_relu(h, g1_ref, b1_ref)
    h = lax.dot_general(h, w2_ref[...], dn,
                        preferred_element_type=jnp.float32)
    o_ref[...] = _bn_relu(h, g2_ref, b2_ref)


@jax.jit
def kernel(x, edge_index, batch, W1, gamma1, beta1, W2, gamma2, beta2, eps):
    del batch
    ei = edge_index.astype(jnp.int32)
    src = ei[0].reshape(_NW, _NCHUNK, _K)
    dst = ei[1].reshape(_NW, _NCHUNK, _K)
    agg = _sc_segment_sum(x, src, dst)
    return pl.pallas_call(
        _mlp_body,
        out_shape=jax.ShapeDtypeStruct((_N_NODES, _D), jnp.float32),
    )(x, agg, W1, gamma1.reshape(1, _D), beta1.reshape(1, _D),
      W2, gamma2.reshape(1, _D), beta2.reshape(1, _D),
      eps.reshape(1, 1).astype(jnp.float32))


# speedup vs baseline: 1.5608x; 1.3814x over previous
import functools

import jax
import jax.numpy as jnp
from jax import lax
from jax.experimental import pallas as pl
from jax.experimental.pallas import tpu as pltpu
from jax.experimental.pallas import tpu_sc as plsc

_N_NODES = 10000
_D = 128
_N_EDGES = 320000
_BN_EPS = 1e-5

_NC = 2
_NS = 16
_NW = _NC * _NS
_EPW = _N_EDGES // _NW
_K = 80
_NCHUNK = _EPW // _K
_NPAD = 10240
_RPT = _NPAD // _NS


@functools.partial(
    pl.kernel,
    out_type=jax.ShapeDtypeStruct((_NC, _NPAD, _D), jnp.float32),
    mesh=plsc.VectorSubcoreMesh(core_axis_name="c", subcore_axis_name="s"),
    scratch_types=[
        pltpu.VMEM((_K, _D), jnp.float32),
        pltpu.VMEM((_K, _D), jnp.float32),
        pltpu.VMEM((_K, _D), jnp.float32),
        pltpu.VMEM((_K, _D), jnp.float32),
        pltpu.VMEM((4, _K), jnp.int32),
        pltpu.VMEM((4, _K), jnp.int32),
        pltpu.VMEM_SHARED((_NPAD, _D), jnp.float32),
        pltpu.SemaphoreType.DMA,
        pltpu.SemaphoreType.DMA,
        pltpu.SemaphoreType.DMA,
        pltpu.SemaphoreType.DMA,
        pltpu.SemaphoreType.DMA,
        pltpu.SemaphoreType.DMA,
        pltpu.SemaphoreType.DMA,
        pltpu.SemaphoreType.DMA,
    ],
)
def _sc_segment_sum(x_hbm, src_hbm, dst_hbm, out_hbm,
                    rows0, rows1, rows2, rows3, sidx, didx, agg_sh,
                    sem0, sem1, sem2, sem3, sems0, sems1, semi0, semi1):
    cid = lax.axis_index("c")
    sid = lax.axis_index("s")
    wid = cid * _NS + sid

    rows = (rows0, rows1, rows2, rows3)
    gsem = (sem0, sem1, sem2, sem3)
    ssem = (sems0, sems1)
    isem = (semi0, semi1)

    def _load_idx(c, s, sem=None):
        sslc = src_hbm.at[wid].at[pl.ds(c, 1)]
        dslc = dst_hbm.at[wid].at[pl.ds(c, 1)]
        if sem is None:
            pltpu.sync_copy(sslc, sidx.at[pl.ds(s, 1)])
            pltpu.sync_copy(dslc, didx.at[pl.ds(s, 1)])
        else:
            pltpu.async_copy(sslc, sidx.at[pl.ds(s, 1)], sem)
            pltpu.async_copy(dslc, didx.at[pl.ds(s, 1)], sem)

    def _wait_idx(c, s, sem):
        pltpu.make_async_copy(src_hbm.at[wid].at[pl.ds(c, 1)],
                              sidx.at[pl.ds(s, 1)], sem).wait()
        pltpu.make_async_copy(dst_hbm.at[wid].at[pl.ds(c, 1)],
                              didx.at[pl.ds(s, 1)], sem).wait()

    zeros16 = jnp.zeros((16,), jnp.float32)

    @pl.loop(0, _K)
    def _(i):
        @pl.loop(0, _D // 16)
        def _(j):
            rows0.at[i, pl.ds(j * 16, 16)][...] = zeros16

    @pl.loop(0, _RPT // _K)
    def _(r):
        pltpu.sync_copy(rows0, agg_sh.at[pl.ds(sid * _RPT + r * _K, _K)])

    plsc.subcore_barrier()

    _load_idx(0, 0)
    _load_idx(1, 1)
    pltpu.async_copy(x_hbm.at[sidx.at[0]], rows[0], gsem[0])
    _load_idx(2, 2, isem[0])
    pltpu.async_copy(x_hbm.at[sidx.at[1]], rows[1], gsem[1])

    def _step(i, k, first=False, g2=True, i3=True):
        p = k % 2
        q = 1 - p
        pltpu.make_async_copy(x_hbm.at[sidx.at[k]], rows[k], gsem[k]).wait()
        if not first:
            pltpu.make_async_copy(
                rows[(k + 3) % 4], agg_sh.at[didx.at[(k + 3) % 4]],
                ssem[q]).wait()
        pltpu.async_copy(rows[k], agg_sh.at[didx.at[k]], ssem[p], add=True)
        if g2:
            k2 = (k + 2) % 4
            _wait_idx(i + 2, k2, isem[k2 % 2])
            pltpu.async_copy(x_hbm.at[sidx.at[k2]], rows[k2], gsem[k2])
        if i3:
            k3 = (k + 3) % 4
            _load_idx(i + 3, k3, isem[k3 % 2])

    _step(0, 0, first=True)
    _step(1, 1)
    _step(2, 2)
    _step(3, 3)

    @pl.loop(1, (_NCHUNK - 5) // 4)
    def _(g):
        _step(4 * g, 0)
        _step(4 * g + 1, 1)
        _step(4 * g + 2, 2)
        _step(4 * g + 3, 3)

    _step(_NCHUNK - 5, 0)
    _step(_NCHUNK - 4, 1)
    _step(_NCHUNK - 3, 2, i3=False)
    _step(_NCHUNK - 2, 3, g2=False, i3=False)
    _step(_NCHUNK - 1, 0, g2=False, i3=False)

    pltpu.make_async_copy(rows[0], agg_sh.at[didx.at[0]], ssem[0]).wait()

    plsc.subcore_barrier()
    pltpu.sync_copy(agg_sh.at[pl.ds(sid * _RPT, _RPT)],
                    out_hbm.at[cid].at[pl.ds(sid * _RPT, _RPT)])


def _mlp_body(x_ref, agg_ref, w1_ref, g1_ref, b1_ref, w2_ref, g2_ref, b2_ref,
              eps_ref, o_ref):
    dn = (((1,), (1,)), ((), ()))
    ones = jnp.ones((1, _N_NODES), jnp.float32)

    def _bn_relu(h, g_ref, b_ref):
        dnc = (((1,), (0,)), ((), ()))
        s = lax.dot_general(ones, h, dnc, preferred_element_type=jnp.float32)
        q = lax.dot_general(ones, h * h, dnc,
                            preferred_element_type=jnp.float32)
        mu = s * (1.0 / _N_NODES)
        var = q * (1.0 / _N_NODES) - mu * mu
        scale = g_ref[...] * lax.rsqrt(var + _BN_EPS)
        return jnp.maximum(scale * (h - mu) + b_ref[...], 0.0)

    h = ((1.0 + eps_ref[0, 0]) * x_ref[...]
         + agg_ref[0, :_N_NODES, :] + agg_ref[1, :_N_NODES, :])
    h = lax.dot_general(h, w1_ref[...], dn,
                        preferred_element_type=jnp.float32)
    h = _bn_relu(h, g1_ref, b1_ref)
    h = lax.dot_general(h, w2_ref[...], dn,
                        preferred_element_type=jnp.float32)
    o_ref[...] = _bn_relu(h, g2_ref, b2_ref)


@jax.jit
def kernel(x, edge_index, batch, W1, gamma1, beta1, W2, gamma2, beta2, eps):
    del batch
    ei = edge_index.astype(jnp.int32)
    src = ei[0].reshape(_NW, _NCHUNK, _K)
    dst = ei[1].reshape(_NW, _NCHUNK, _K)
    agg = _sc_segment_sum(x, src, dst)
    return pl.pallas_call(
        _mlp_body,
        out_shape=jax.ShapeDtypeStruct((_N_NODES, _D), jnp.float32),
    )(x, agg, W1, gamma1.reshape(1, _D), beta1.reshape(1, _D),
      W2, gamma2.reshape(1, _D), beta2.reshape(1, _D),
      eps.reshape(1, 1).astype(jnp.float32))


# speedup vs baseline: 1.5783x; 1.0112x over previous
import functools

import jax
import jax.numpy as jnp
from jax import lax
from jax.experimental import pallas as pl
from jax.experimental.pallas import tpu as pltpu
from jax.experimental.pallas import tpu_sc as plsc

_N_NODES = 10000
_D = 128
_N_EDGES = 320000
_BN_EPS = 1e-5

_NC = 2
_NS = 16
_NW = _NC * _NS
_EPW = _N_EDGES // _NW
_K = 80
_NCHUNK = _EPW // _K
_NPAD = 10240
_RPT = _NPAD // _NS


@functools.partial(
    pl.kernel,
    out_type=jax.ShapeDtypeStruct((_NC, _NPAD, _D), jnp.float32),
    mesh=plsc.VectorSubcoreMesh(core_axis_name="c", subcore_axis_name="s"),
    scratch_types=[
        pltpu.VMEM((_K, _D), jnp.float32),
        pltpu.VMEM((_K, _D), jnp.float32),
        pltpu.VMEM((_K, _D), jnp.float32),
        pltpu.VMEM((_K, _D), jnp.float32),
        pltpu.VMEM((8, _K), jnp.int32),
        pltpu.VMEM((8, _K), jnp.int32),
        pltpu.VMEM_SHARED((_NPAD, _D), jnp.float32),
        pltpu.SemaphoreType.DMA,
        pltpu.SemaphoreType.DMA,
        pltpu.SemaphoreType.DMA,
        pltpu.SemaphoreType.DMA,
        pltpu.SemaphoreType.DMA,
        pltpu.SemaphoreType.DMA,
        pltpu.SemaphoreType.DMA,
        pltpu.SemaphoreType.DMA,
    ],
)
def _sc_segment_sum(x_hbm, src_hbm, dst_hbm, out_hbm,
                    rows0, rows1, rows2, rows3, sidx, didx, agg_sh,
                    sem0, sem1, sem2, sem3, sems0, sems1, semi0, semi1):
    cid = lax.axis_index("c")
    sid = lax.axis_index("s")
    wid = cid * _NS + sid

    rows = (rows0, rows1, rows2, rows3)
    gsem = (sem0, sem1, sem2, sem3)
    ssem = (sems0, sems1)
    isem = (semi0, semi1)

    def _load_idx(c, s, sem=None):
        sslc = src_hbm.at[wid].at[pl.ds(c, 1)]
        dslc = dst_hbm.at[wid].at[pl.ds(c, 1)]
        if sem is None:
            pltpu.sync_copy(sslc, sidx.at[pl.ds(s, 1)])
            pltpu.sync_copy(dslc, didx.at[pl.ds(s, 1)])
        else:
            pltpu.async_copy(sslc, sidx.at[pl.ds(s, 1)], sem)
            pltpu.async_copy(dslc, didx.at[pl.ds(s, 1)], sem)

    def _wait_idx(c, s, sem):
        pltpu.make_async_copy(src_hbm.at[wid].at[pl.ds(c, 1)],
                              sidx.at[pl.ds(s, 1)], sem).wait()
        pltpu.make_async_copy(dst_hbm.at[wid].at[pl.ds(c, 1)],
                              didx.at[pl.ds(s, 1)], sem).wait()

    zeros16 = jnp.zeros((16,), jnp.float32)

    @pl.loop(0, _K)
    def _(i):
        @pl.loop(0, _D // 16)
        def _(j):
            rows0.at[i, pl.ds(j * 16, 16)][...] = zeros16

    @pl.loop(0, _RPT // _K)
    def _(r):
        pltpu.sync_copy(rows0, agg_sh.at[pl.ds(sid * _RPT + r * _K, _K)])

    plsc.subcore_barrier()

    _load_idx(0, 0)
    _load_idx(1, 1)
    _load_idx(2, 2)
    pltpu.async_copy(x_hbm.at[sidx.at[0]], rows[0], gsem[0])
    pltpu.async_copy(x_hbm.at[sidx.at[1]], rows[1], gsem[1])
    pltpu.async_copy(x_hbm.at[sidx.at[2]], rows[2], gsem[2])
    _load_idx(3, 3, isem[1])

    def _step(i, j, first=False, g3=True, i4=True):
        k = j % 4
        p = j % 2
        q = 1 - p
        pltpu.make_async_copy(x_hbm.at[sidx.at[j]], rows[k], gsem[k]).wait()
        if not first:
            pltpu.make_async_copy(
                rows[(k + 3) % 4], agg_sh.at[didx.at[(j + 7) % 8]],
                ssem[q]).wait()
        pltpu.async_copy(rows[k], agg_sh.at[didx.at[j]], ssem[p], add=True)
        if g3:
            j3 = (j + 3) % 8
            _wait_idx(i + 3, j3, isem[j3 % 2])
            pltpu.async_copy(x_hbm.at[sidx.at[j3]], rows[(k + 3) % 4],
                             gsem[(k + 3) % 4])
        if i4:
            j4 = (j + 4) % 8
            _load_idx(i + 4, j4, isem[j4 % 2])

    _step(0, 0, first=True)
    for _j in range(1, 8):
        _step(_j, _j)

    @pl.loop(1, (_NCHUNK - 5) // 8)
    def _(g):
        for _j in range(8):
            _step(8 * g + _j, _j)

    _step(_NCHUNK - 5, 0)
    _step(_NCHUNK - 4, 1, i4=False)
    _step(_NCHUNK - 3, 2, g3=False, i4=False)
    _step(_NCHUNK - 2, 3, g3=False, i4=False)
    _step(_NCHUNK - 1, 4, g3=False, i4=False)

    pltpu.make_async_copy(rows[0], agg_sh.at[didx.at[4]], ssem[0]).wait()

    plsc.subcore_barrier()
    pltpu.sync_copy(agg_sh.at[pl.ds(sid * _RPT, _RPT)],
                    out_hbm.at[cid].at[pl.ds(sid * _RPT, _RPT)])


def _mlp_body(x_ref, agg_ref, w1_ref, g1_ref, b1_ref, w2_ref, g2_ref, b2_ref,
              eps_ref, o_ref):
    dn = (((1,), (1,)), ((), ()))
    ones = jnp.ones((1, _N_NODES), jnp.float32)

    def _bn_relu(h, g_ref, b_ref):
        dnc = (((1,), (0,)), ((), ()))
        s = lax.dot_general(ones, h, dnc, preferred_element_type=jnp.float32)
        q = lax.dot_general(ones, h * h, dnc,
                            preferred_element_type=jnp.float32)
        mu = s * (1.0 / _N_NODES)
        var = q * (1.0 / _N_NODES) - mu * mu
        scale = g_ref[...] * lax.rsqrt(var + _BN_EPS)
        return jnp.maximum(scale * (h - mu) + b_ref[...], 0.0)

    h = ((1.0 + eps_ref[0, 0]) * x_ref[...]
         + agg_ref[0, :_N_NODES, :] + agg_ref[1, :_N_NODES, :])
    h = lax.dot_general(h, w1_ref[...], dn,
                        preferred_element_type=jnp.float32)
    h = _bn_relu(h, g1_ref, b1_ref)
    h = lax.dot_general(h, w2_ref[...], dn,
                        preferred_element_type=jnp.float32)
    o_ref[...] = _bn_relu(h, g2_ref, b2_ref)


@jax.jit
def kernel(x, edge_index, batch, W1, gamma1, beta1, W2, gamma2, beta2, eps):
    del batch
    ei = edge_index.astype(jnp.int32)
    src = ei[0].reshape(_NW, _NCHUNK, _K)
    dst = ei[1].reshape(_NW, _NCHUNK, _K)
    agg = _sc_segment_sum(x, src, dst)
    return pl.pallas_call(
        _mlp_body,
        out_shape=jax.ShapeDtypeStruct((_N_NODES, _D), jnp.float32),
    )(x, agg, W1, gamma1.reshape(1, _D), beta1.reshape(1, _D),
      W2, gamma2.reshape(1, _D), beta2.reshape(1, _D),
      eps.reshape(1, 1).astype(jnp.float32))
